# baseline (speedup 1.0000x reference)
import functools

import jax
import jax.numpy as jnp
from jax import lax
from jax.experimental import pallas as pl
from jax.experimental.pallas import tpu as pltpu
from jax.experimental.pallas import tpu_sc as plsc

B, T, ENC_DIM = 16, 4096, 512
N_BINS = 256
NTOT = B * T
NC, NS, L = 2, 16, 16
NW = NC * NS
PER_W = NTOT // NW
CHUNK = 32
NCHUNK = PER_W // CHUNK


NBUF = 4
LOOKAHEAD = 2


def _sc_body(cls_hbm, table_hbm, out_hbm, table_sh, cls_sh, cls_sm,
             rows0, rows1, rows2, rows3,
             rsem, wsem0, wsem1, wsem2, wsem3):
    cid = lax.axis_index("c")
    sid = lax.axis_index("s")
    wid = sid * NC + cid
    base = wid * PER_W

    @pl.when(sid == 0)
    def _():
        pltpu.sync_copy(table_hbm, table_sh)
        pltpu.sync_copy(cls_hbm, cls_sh)

    plsc.subcore_barrier()

    bufs = (rows0, rows1, rows2, rows3)
    wsems = (wsem0, wsem1, wsem2, wsem3)

    def write(g, b):
        return pltpu.make_async_copy(
            bufs[b], out_hbm.at[pl.ds(base + g * CHUNK, CHUNK)], wsems[b])

    def chunk_body(gq, _):
        for bq in range(NBUF):
            g = gq * NBUF + bq
            buf = bufs[bq]
            pltpu.sync_copy(cls_sh.at[pl.ds(base + g * CHUNK, CHUNK)], cls_sm)

            @pl.when(g >= NBUF)
            def _():
                write(g - NBUF, bq).wait()

            def row_start(r, _):
                c = cls_sm[r]
                pltpu.make_async_copy(
                    table_sh.at[pl.ds(c, 1)], buf.at[pl.ds(r, 1)],
                    rsem).start()
                return 0

            lax.fori_loop(0, CHUNK, row_start, 0)

            def row_drain(r, _):
                pltpu.make_async_copy(
                    table_sh.at[pl.ds(0, 1)], buf.at[pl.ds(r, 1)],
                    rsem).wait()
                return 0

            lax.fori_loop(0, CHUNK, row_drain, 0)
            write(g, bq).start()
        return 0

    lax.fori_loop(0, NCHUNK // NBUF, chunk_body, 0)
    for g in range(NCHUNK - NBUF, NCHUNK):
        write(g, g % NBUF).wait()


_sc_lookup = functools.partial(
    pl.kernel,
    out_type=jax.ShapeDtypeStruct((NTOT, ENC_DIM), jnp.float32),
    mesh=plsc.VectorSubcoreMesh(core_axis_name="c", subcore_axis_name="s",
                                num_cores=NC, num_subcores=NS),
    scratch_types=[
        pltpu.VMEM_SHARED((N_BINS, ENC_DIM), jnp.float32),
        pltpu.VMEM_SHARED((NTOT,), jnp.int32),
        pltpu.SMEM((CHUNK,), jnp.int32),
        pltpu.VMEM((CHUNK, ENC_DIM), jnp.float32),
        pltpu.VMEM((CHUNK, ENC_DIM), jnp.float32),
        pltpu.VMEM((CHUNK, ENC_DIM), jnp.float32),
        pltpu.VMEM((CHUNK, ENC_DIM), jnp.float32),
        pltpu.SemaphoreType.DMA,
        pltpu.SemaphoreType.DMA,
        pltpu.SemaphoreType.DMA,
        pltpu.SemaphoreType.DMA,
        pltpu.SemaphoreType.DMA,
    ],
)(_sc_body)


_CLS_BT = 8192


def _tc_cls_body(t_ref, bins_ref, ones_ref, c_ref):
    maskf = (bins_ref[...] < t_ref[...]).astype(jnp.bfloat16)
    c_ref[...] = jnp.dot(maskf, ones_ref[...],
                         preferred_element_type=jnp.float32).astype(jnp.int32)


def _tc_classes(t2d, bins_row, ones8):
    return pl.pallas_call(
        _tc_cls_body,
        grid=(NTOT // _CLS_BT,),
        in_specs=[
            pl.BlockSpec((_CLS_BT, 1), lambda i: (i, 0)),
            pl.BlockSpec((1, N_BINS), lambda i: (0, 0)),
            pl.BlockSpec((N_BINS, 8), lambda i: (0, 0)),
        ],
        out_specs=pl.BlockSpec((_CLS_BT, 8), lambda i: (i, 0)),
        out_shape=jax.ShapeDtypeStruct((NTOT, 8), jnp.int32),
    )(t2d, bins_row, ones8)


_MV_CHUNK = 2048
_MV_NBUF = 4


def _tc_pred_body(f_hbm, w_ref, b_ref, o_ref, buf0, buf1, buf2, buf3,
                  sem0, sem1, sem2, sem3):
    bufs = (buf0, buf1, buf2, buf3)
    sems = (sem0, sem1, sem2, sem3)
    nch = NTOT // _MV_CHUNK

    def fetch(k, b):
        return pltpu.make_async_copy(
            f_hbm.at[pl.ds(k * _MV_CHUNK, _MV_CHUNK)], bufs[b], sems[b])

    for k in range(_MV_NBUF):
        fetch(k, k).start()

    bias = b_ref[0, 0]

    def body(kq, _):
        for bq in range(_MV_NBUF):
            k = kq * _MV_NBUF + bq
            fetch(k, bq).wait()
            o_ref[pl.ds(k * _MV_CHUNK, _MV_CHUNK), :] = jnp.dot(
                bufs[bq][...], w_ref[...],
                preferred_element_type=jnp.float32) + bias

            @pl.when(k + _MV_NBUF < nch)
            def _():
                fetch(k + _MV_NBUF, bq).start()
        return 0

    lax.fori_loop(0, nch // _MV_NBUF, body, 0)


def _tc_pred(frames2d, w8, b2d):
    return pl.pallas_call(
        _tc_pred_body,
        in_specs=[
            pl.BlockSpec(memory_space=pltpu.MemorySpace.HBM),
            pl.BlockSpec(memory_space=pltpu.MemorySpace.VMEM),
            pl.BlockSpec(memory_space=pltpu.MemorySpace.VMEM),
        ],
        out_specs=pl.BlockSpec(memory_space=pltpu.MemorySpace.VMEM),
        out_shape=jax.ShapeDtypeStruct((NTOT, 8), jnp.float32),
        scratch_shapes=[
            pltpu.VMEM((_MV_CHUNK, ENC_DIM), jnp.float32),
            pltpu.VMEM((_MV_CHUNK, ENC_DIM), jnp.float32),
            pltpu.VMEM((_MV_CHUNK, ENC_DIM), jnp.float32),
            pltpu.VMEM((_MV_CHUNK, ENC_DIM), jnp.float32),
            pltpu.SemaphoreType.DMA,
            pltpu.SemaphoreType.DMA,
            pltpu.SemaphoreType.DMA,
            pltpu.SemaphoreType.DMA,
        ],
    )(frames2d, w8, b2d)


def kernel(frames, target, W_pred, b_pred, emb_table, bins):
    bins_row = jnp.concatenate(
        [bins, jnp.full((1,), jnp.inf, jnp.float32)]).reshape(1, N_BINS)
    ones8 = jnp.ones((N_BINS, 8), jnp.bfloat16)
    classes8 = _tc_classes(target.reshape(NTOT, 1), bins_row, ones8)
    classes = classes8[:, 0]

    emb_flat = _sc_lookup(classes, emb_table)
    emb = emb_flat.reshape(B, T, ENC_DIM)

    frames2d = frames.reshape(NTOT, ENC_DIM)
    w8 = jnp.concatenate(
        [W_pred, jnp.zeros((ENC_DIM, 7), jnp.float32)], axis=1)
    pred8 = _tc_pred(frames2d, w8, b_pred.reshape(1, 1))
    prediction = pred8[:, 0].reshape(B, T)
    return (prediction, emb)


# speedup vs baseline: 1.0159x; 1.0159x over previous
import functools

import jax
import jax.numpy as jnp
from jax import lax
from jax.experimental import pallas as pl
from jax.experimental.pallas import tpu as pltpu
from jax.experimental.pallas import tpu_sc as plsc

B, T, ENC_DIM = 16, 4096, 512
N_BINS = 256
NTOT = B * T
NC, NS, L = 2, 16, 16
NW = NC * NS
PER_W = NTOT // NW
CHUNK = 32
NCHUNK = PER_W // CHUNK


NBUF = 4
LOOKAHEAD = 2


def _sc_body(cls_hbm, table_hbm, out_hbm, table_sh, cls_sh, cls_sm,
             rows0, rows1, rows2, rows3,
             rsem, wsem0, wsem1, wsem2, wsem3):
    cid = lax.axis_index("c")
    sid = lax.axis_index("s")
    wid = sid * NC + cid
    base = wid * PER_W

    @pl.when(sid == 0)
    def _():
        pltpu.sync_copy(table_hbm, table_sh)
        pltpu.sync_copy(cls_hbm, cls_sh)

    plsc.subcore_barrier()

    bufs = (rows0, rows1, rows2, rows3)
    wsems = (wsem0, wsem1, wsem2, wsem3)

    def write(g, b):
        return pltpu.make_async_copy(
            bufs[b], out_hbm.at[pl.ds(base + g * CHUNK, CHUNK)], wsems[b])

    def chunk_body(gq, _):
        for bq in range(NBUF):
            g = gq * NBUF + bq
            buf = bufs[bq]
            pltpu.sync_copy(cls_sh.at[pl.ds(base + g * CHUNK, CHUNK)], cls_sm)

            @pl.when(g >= NBUF)
            def _():
                write(g - NBUF, bq).wait()

            def row_start(r, _):
                c = cls_sm[r]
                pltpu.make_async_copy(
                    table_sh.at[pl.ds(c, 1)], buf.at[pl.ds(r, 1)],
                    rsem).start()
                return 0

            lax.fori_loop(0, CHUNK, row_start, 0)

            def row_drain(r, _):
                pltpu.make_async_copy(
                    table_sh.at[pl.ds(0, 1)], buf.at[pl.ds(r, 1)],
                    rsem).wait()
                return 0

            lax.fori_loop(0, CHUNK, row_drain, 0)
            write(g, bq).start()
        return 0

    lax.fori_loop(0, NCHUNK // NBUF, chunk_body, 0)
    for g in range(NCHUNK - NBUF, NCHUNK):
        write(g, g % NBUF).wait()


_sc_lookup = functools.partial(
    pl.kernel,
    out_type=jax.ShapeDtypeStruct((NTOT, ENC_DIM), jnp.float32),
    mesh=plsc.VectorSubcoreMesh(core_axis_name="c", subcore_axis_name="s",
                                num_cores=NC, num_subcores=NS),
    scratch_types=[
        pltpu.VMEM_SHARED((N_BINS, ENC_DIM), jnp.float32),
        pltpu.VMEM_SHARED((NTOT,), jnp.int32),
        pltpu.SMEM((CHUNK,), jnp.int32),
        pltpu.VMEM((CHUNK, ENC_DIM), jnp.float32),
        pltpu.VMEM((CHUNK, ENC_DIM), jnp.float32),
        pltpu.VMEM((CHUNK, ENC_DIM), jnp.float32),
        pltpu.VMEM((CHUNK, ENC_DIM), jnp.float32),
        pltpu.SemaphoreType.DMA,
        pltpu.SemaphoreType.DMA,
        pltpu.SemaphoreType.DMA,
        pltpu.SemaphoreType.DMA,
        pltpu.SemaphoreType.DMA,
    ],
)(_sc_body)


_CLS_BT = 8192


def _tc_cls_body(t_ref, bins_ref, ones_ref, c_ref):
    maskf = (bins_ref[...] < t_ref[...]).astype(jnp.bfloat16)
    c_ref[...] = jnp.dot(maskf, ones_ref[...],
                         preferred_element_type=jnp.float32).astype(jnp.int32)


def _tc_classes(t2d, bins_row, ones8):
    return pl.pallas_call(
        _tc_cls_body,
        grid=(NTOT // _CLS_BT,),
        in_specs=[
            pl.BlockSpec((_CLS_BT, 1), lambda i: (i, 0)),
            pl.BlockSpec((1, N_BINS), lambda i: (0, 0)),
            pl.BlockSpec((N_BINS, 8), lambda i: (0, 0)),
        ],
        out_specs=pl.BlockSpec((_CLS_BT, 8), lambda i: (i, 0)),
        out_shape=jax.ShapeDtypeStruct((NTOT, 8), jnp.int32),
    )(t2d, bins_row, ones8)


_TC_BT = 4096


def _tc_pred_body(f_ref, w_ref, b_ref, o_ref):
    o_ref[...] = jnp.dot(f_ref[...], w_ref[...],
                         preferred_element_type=jnp.float32) + b_ref[0, 0]


def _tc_pred(frames2d, w8, b2d):
    return pl.pallas_call(
        _tc_pred_body,
        grid=(NTOT // _TC_BT,),
        in_specs=[
            pl.BlockSpec((_TC_BT, ENC_DIM), lambda i: (i, 0)),
            pl.BlockSpec((ENC_DIM, 8), lambda i: (0, 0)),
            pl.BlockSpec((1, 1), lambda i: (0, 0)),
        ],
        out_specs=pl.BlockSpec((_TC_BT, 8), lambda i: (i, 0)),
        out_shape=jax.ShapeDtypeStruct((NTOT, 8), jnp.float32),
    )(frames2d, w8, b2d)


def kernel(frames, target, W_pred, b_pred, emb_table, bins):
    bins_row = jnp.concatenate(
        [bins, jnp.full((1,), jnp.inf, jnp.float32)]).reshape(1, N_BINS)
    ones8 = jnp.ones((N_BINS, 8), jnp.bfloat16)
    classes8 = _tc_classes(target.reshape(NTOT, 1), bins_row, ones8)
    classes = classes8[:, 0]

    emb_flat = _sc_lookup(classes, emb_table)
    emb = emb_flat.reshape(B, T, ENC_DIM)

    frames2d = frames.reshape(NTOT, ENC_DIM)
    w8 = jnp.concatenate(
        [W_pred, jnp.zeros((ENC_DIM, 7), jnp.float32)], axis=1)
    pred8 = _tc_pred(frames2d, w8, b_pred.reshape(1, 1))
    prediction = pred8[:, 0].reshape(B, T)
    return (prediction, emb)
